# baseline (speedup 1.0000x reference)
import jax
import jax.numpy as jnp
from jax import lax
from jax.experimental import pallas as pl
from jax.experimental.pallas import tpu as pltpu
from jax.experimental.pallas import tpu_sc as plsc

_NC = 2
_NS = 16
_NW = _NC * _NS
_L = 16


def _sc_lookup_t(table, ids_t):
    s_len, b_len = ids_t.shape
    d = table.shape[1]
    assert b_len == _NW * 128 and d % 8 == 0

    mesh = plsc.VectorSubcoreMesh(core_axis_name="c", subcore_axis_name="s",
                                  num_cores=_NC, num_subcores=_NS)

    @pl.kernel(
        out_type=jax.ShapeDtypeStruct((s_len, d // 8, _NW, 8, 128), jnp.float32),
        mesh=mesh,
        scratch_types=[
            pltpu.VMEM((s_len, 128), jnp.int32),
            pltpu.VMEM((4, 128, d), jnp.float32),
            pltpu.VMEM((2, d // 8, 8, 128), jnp.float32),
            pltpu.SemaphoreType.DMA((4,)),
            pltpu.SemaphoreType.DMA((2,)),
        ],
        compiler_params=pltpu.CompilerParams(use_tc_tiling_on_sc=False,
                                             needs_layout_passes=False),
    )
    def k(table_hbm, ids_hbm, out_hbm, idx_v, grow, tbuf, gsem, osem):
        wid = lax.axis_index("s") * _NC + lax.axis_index("c")
        pltpu.sync_copy(ids_hbm.at[:, pl.ds(wid * 128, 128)], idx_v)

        row_base = [
            (lax.iota(jnp.int32, _L) + _L * kk) for kk in range(128 // _L)
        ]

        def fire_gather(s, buf):
            pltpu.async_copy(
                table_hbm.at[idx_v.at[s]], grow.at[buf], gsem.at[buf]
            )

        def drain_gather(buf):
            pltpu.make_async_copy(
                table_hbm.at[idx_v.at[0]], grow.at[buf], gsem.at[buf]
            ).wait()

        def fire_out(s, buf):
            pltpu.async_copy(
                tbuf.at[buf], out_hbm.at[s, :, wid], osem.at[buf]
            )

        def drain_out(s, buf):
            pltpu.make_async_copy(
                tbuf.at[buf], out_hbm.at[s, :, wid], osem.at[buf]
            ).wait()

        for p in range(3):
            fire_gather(p, p)

        def transpose(gbuf, tb):
            def gat(dd):
                col = jnp.full((_L,), dd, jnp.int32)
                return [
                    plsc.load_gather(grow.at[gbuf], [row_base[kk], col])
                    for kk in range(128 // _L)
                ]

            vs = gat(0)
            for dd in range(d):
                nxt = gat(dd + 1) if dd + 1 < d else None
                for kk in range(128 // _L):
                    tbuf[tb, dd // 8, dd % 8, pl.ds(kk * _L, _L)] = vs[kk]
                vs = nxt

        def step(i, carry):
            s0 = i * 4
            for j in range(4):
                s = s0 + j
                gbuf = j
                tb = j % 2
                drain_gather(gbuf)

                @pl.when(s + 3 < s_len)
                def _():
                    fire_gather(s + 3, (j + 3) % 4)

                @pl.when(s >= 2)
                def _():
                    drain_out(s - 2, tb)

                transpose(gbuf, tb)
                fire_out(s, tb)
            return carry

        lax.fori_loop(0, s_len // 4, step, 0)
        drain_out(s_len - 2, lax.rem(s_len - 2, 2))
        drain_out(s_len - 1, lax.rem(s_len - 1, 2))

    return k(table, ids_t)


def kernel(input_ids, frozen_table, trainable_table):
    nb, ns = input_ids.shape
    d = frozen_table.shape[-1]
    table = jnp.concatenate([frozen_table, trainable_table], axis=0)
    ids_t = input_ids.astype(jnp.int32).T
    out5 = _sc_lookup_t(table, ids_t)
    return out5.transpose(2, 4, 0, 1, 3).reshape(nb, ns, d)


# speedup vs baseline: 1.8575x; 1.8575x over previous
import jax
import jax.numpy as jnp
from jax import lax
from jax.experimental import pallas as pl
from jax.experimental.pallas import tpu as pltpu
from jax.experimental.pallas import tpu_sc as plsc

_NC = 2
_NS = 16
_NW = _NC * _NS
_L = 16


def _sc_lookup_t(table, ids_t, d):
    s_len, b_len = ids_t.shape
    dp = table.shape[1]
    assert b_len == _NW * 128 and d % 8 == 0

    mesh = plsc.VectorSubcoreMesh(core_axis_name="c", subcore_axis_name="s",
                                  num_cores=_NC, num_subcores=_NS)

    @pl.kernel(
        out_type=jax.ShapeDtypeStruct((s_len, d // 8, _NW, 8, 128), jnp.float32),
        mesh=mesh,
        scratch_types=[
            pltpu.VMEM((s_len, 128), jnp.int32),
            pltpu.VMEM((4, 128, dp), jnp.float32),
            pltpu.VMEM((2, d // 8, 8, 129), jnp.float32),
            pltpu.SemaphoreType.DMA((4,)),
            pltpu.SemaphoreType.DMA((2,)),
        ],
        compiler_params=pltpu.CompilerParams(use_tc_tiling_on_sc=False,
                                             needs_layout_passes=False),
    )
    def k(table_hbm, ids_hbm, out_hbm, idx_v, grow, tbuf, gsem, osem):
        wid = lax.axis_index("s") * _NC + lax.axis_index("c")
        pltpu.sync_copy(ids_hbm.at[:, pl.ds(wid * 128, 128)], idx_v)

        d_hi = [(lax.iota(jnp.int32, _L) + _L * m) // 8 for m in range(d // _L)]
        d_lo = [
            lax.rem(lax.iota(jnp.int32, _L) + _L * m, 8) for m in range(d // _L)
        ]

        def fire_gather(s, buf):
            pltpu.async_copy(
                table_hbm.at[idx_v.at[s]], grow.at[buf], gsem.at[buf]
            )

        def drain_gather(buf):
            pltpu.make_async_copy(
                table_hbm.at[idx_v.at[0]], grow.at[buf], gsem.at[buf]
            ).wait()

        def fire_out(s, buf):
            pltpu.async_copy(
                tbuf.at[buf, :, :, pl.ds(0, 128)], out_hbm.at[s, :, wid],
                osem.at[buf],
            )

        def drain_out(s, buf):
            pltpu.make_async_copy(
                tbuf.at[buf, :, :, pl.ds(0, 128)], out_hbm.at[s, :, wid],
                osem.at[buf],
            ).wait()

        for p in range(3):
            fire_gather(p, p)

        def transpose(gbuf, tb):
            for b in range(128):
                col = jnp.full((_L,), b, jnp.int32)
                for m in range(d // _L):
                    v = grow[gbuf, b, pl.ds(m * _L, _L)]
                    plsc.store_scatter(
                        tbuf.at[tb], [d_hi[m], d_lo[m], col], v
                    )

        def step(i, carry):
            s0 = i * 4
            for j in range(4):
                s = s0 + j
                gbuf = j
                tb = j % 2
                drain_gather(gbuf)

                @pl.when(s + 3 < s_len)
                def _():
                    fire_gather(s + 3, (j + 3) % 4)

                @pl.when(s >= 2)
                def _():
                    drain_out(s - 2, tb)

                transpose(gbuf, tb)
                fire_out(s, tb)
            return carry

        lax.fori_loop(0, s_len // 4, step, 0)
        drain_out(s_len - 2, lax.rem(s_len - 2, 2))
        drain_out(s_len - 1, lax.rem(s_len - 1, 2))

    return k(table, ids_t)


def kernel(input_ids, frozen_table, trainable_table):
    nb, ns = input_ids.shape
    d = frozen_table.shape[-1]
    table = jnp.concatenate([frozen_table, trainable_table], axis=0)
    ids_t = input_ids.astype(jnp.int32).T
    out5 = _sc_lookup_t(table, ids_t, d)
    return out5.transpose(2, 4, 0, 1, 3).reshape(nb, ns, d)


# speedup vs baseline: 1.8873x; 1.0160x over previous
import jax
import jax.numpy as jnp
from jax import lax
from jax.experimental import pallas as pl
from jax.experimental.pallas import tpu as pltpu
from jax.experimental.pallas import tpu_sc as plsc

_NC = 2
_NS = 16
_NW = _NC * _NS
_L = 16


def _sc_lookup_t(table, ids_t, d):
    s_len, b_len = ids_t.shape
    dp = table.shape[1]
    assert b_len == _NW * 128 and d % 8 == 0

    mesh = plsc.VectorSubcoreMesh(core_axis_name="c", subcore_axis_name="s",
                                  num_cores=_NC, num_subcores=_NS)

    @pl.kernel(
        out_type=jax.ShapeDtypeStruct((s_len, d // 8, _NW, 8, 128), jnp.float32),
        mesh=mesh,
        scratch_types=[
            pltpu.VMEM((s_len, 128), jnp.int32),
            pltpu.VMEM((4, 128, dp), jnp.float32),
            pltpu.VMEM((2, d // 8, 8, 129), jnp.float32),
            pltpu.SemaphoreType.DMA((4,)),
            pltpu.SemaphoreType.DMA((2,)),
        ],
        compiler_params=pltpu.CompilerParams(use_tc_tiling_on_sc=False,
                                             needs_layout_passes=False),
    )
    def k(table_hbm, ids_hbm, out_hbm, idx_v, grow, tbuf, gsem, osem):
        wid = lax.axis_index("s") * _NC + lax.axis_index("c")
        pltpu.sync_copy(ids_hbm.at[:, pl.ds(wid * 128, 128)], idx_v)

        d_hi = [(lax.iota(jnp.int32, _L) + _L * m) // 8 for m in range(d // _L)]
        d_lo = [
            lax.rem(lax.iota(jnp.int32, _L) + _L * m, 8) for m in range(d // _L)
        ]

        def fire_gather(s, buf):
            pltpu.async_copy(
                table_hbm.at[idx_v.at[s]], grow.at[buf], gsem.at[buf]
            )

        def drain_gather(buf):
            pltpu.make_async_copy(
                table_hbm.at[idx_v.at[0]], grow.at[buf], gsem.at[buf]
            ).wait()

        def fire_out(s, buf):
            pltpu.async_copy(
                tbuf.at[buf, :, :, pl.ds(0, 128)], out_hbm.at[s, :, wid],
                osem.at[buf],
            )

        def drain_out(s, buf):
            pltpu.make_async_copy(
                tbuf.at[buf, :, :, pl.ds(0, 128)], out_hbm.at[s, :, wid],
                osem.at[buf],
            ).wait()

        for p in range(3):
            fire_gather(p, p)

        one = jnp.full((_L,), 1, jnp.int32)

        def transpose(gbuf, tb):
            nm = d // _L

            def loads(b):
                return [grow[gbuf, b, pl.ds(m * _L, _L)] for m in range(nm)]

            col = jnp.full((_L,), 0, jnp.int32)
            vs = loads(0)
            for b in range(128):
                nxt = loads(b + 1) if b + 1 < 128 else None
                for m in range(nm):
                    plsc.store_scatter(tbuf.at[tb], [d_hi[m], d_lo[m], col], vs[m])
                col = col + one
                vs = nxt

        def step(i, carry):
            s0 = i * 4
            for j in range(4):
                s = s0 + j
                gbuf = j
                tb = j % 2
                drain_gather(gbuf)

                @pl.when(s + 3 < s_len)
                def _():
                    fire_gather(s + 3, (j + 3) % 4)

                @pl.when(s >= 2)
                def _():
                    drain_out(s - 2, tb)

                transpose(gbuf, tb)
                fire_out(s, tb)
            return carry

        lax.fori_loop(0, s_len // 4, step, 0)
        drain_out(s_len - 2, lax.rem(s_len - 2, 2))
        drain_out(s_len - 1, lax.rem(s_len - 1, 2))

    return k(table, ids_t)


def kernel(input_ids, frozen_table, trainable_table):
    nb, ns = input_ids.shape
    d = frozen_table.shape[-1]
    table = jnp.concatenate([frozen_table, trainable_table], axis=0)
    ids_t = input_ids.astype(jnp.int32).T
    out5 = _sc_lookup_t(table, ids_t, d)
    return out5.transpose(2, 4, 0, 1, 3).reshape(nb, ns, d)
